# baseline (speedup 1.0000x reference)
import functools

import jax
import jax.numpy as jnp
from jax import lax
from jax.experimental import pallas as pl
from jax.experimental.pallas import tpu as pltpu
from jax.experimental.pallas import tpu_sc as plsc

BATCH = 16384
DIM = 32
NUM_CORES = 2
NUM_SUBCORES = 16
NUM_WORKERS = NUM_CORES * NUM_SUBCORES
B_PER_W = BATCH // NUM_WORKERS
CHUNK = 128
N_CHUNKS = B_PER_W // CHUNK
LANES = 16


def _sc_body(thetaT_hbm, u_hbm, v_hbm, out_hbm,
             idx_u, idx_v, gu, gv, acc_v, sem):
    wid = lax.axis_index("s") * NUM_CORES + lax.axis_index("c")
    base = wid * B_PER_W

    pltpu.sync_copy(u_hbm.at[wid], idx_u)
    pltpu.sync_copy(v_hbm.at[wid], idx_v)

    pending = []
    for d in range(DIM):
        fired = []
        for c in range(N_CHUNKS):
            fired.append(pltpu.async_copy(
                thetaT_hbm.at[d].at[idx_u.at[c]],
                gu.at[d, pl.ds(c * CHUNK, CHUNK)], sem))
            fired.append(pltpu.async_copy(
                thetaT_hbm.at[d].at[idx_v.at[c]],
                gv.at[d, pl.ds(c * CHUNK, CHUNK)], sem))
        for cp in pending:
            cp.wait()
        pending = fired
    for cp in pending:
        cp.wait()

    def compute_body(g, carry):
        sl = pl.ds(g * LANES, LANES)
        acc = gu[0, sl] * gv[0, sl]
        for d in range(1, DIM):
            acc = acc - gu[d, sl] * gv[d, sl]
        acc_v[sl] = acc
        return carry

    lax.fori_loop(0, B_PER_W // LANES, compute_body, 0)

    pltpu.sync_copy(acc_v, out_hbm.at[pl.ds(base, B_PER_W)])


def _sc_lorentz(thetaT, u3, v3):
    mesh = plsc.VectorSubcoreMesh(core_axis_name="c", subcore_axis_name="s")
    k = pl.kernel(
        _sc_body,
        out_type=jax.ShapeDtypeStruct((BATCH,), jnp.float32),
        mesh=mesh,
        compiler_params=pltpu.CompilerParams(
            needs_layout_passes=False, use_tc_tiling_on_sc=False),
        scratch_types=[
            pltpu.VMEM((N_CHUNKS, CHUNK), jnp.int32),
            pltpu.VMEM((N_CHUNKS, CHUNK), jnp.int32),
            pltpu.VMEM((DIM, B_PER_W), jnp.float32),
            pltpu.VMEM((DIM, B_PER_W), jnp.float32),
            pltpu.VMEM((B_PER_W,), jnp.float32),
            pltpu.SemaphoreType.DMA,
        ],
    )
    return k(thetaT, u3, v3)


def _tc_body(negl_ref, r_ref, t_ref, o_ref):
    w = jnp.clip(negl_ref[...], 1.0 + 1e-6, 100.0)
    duv = jnp.log(w + jnp.sqrt((w - 1.0) * (w + 1.0)))
    o_ref[...] = 1.0 / (jnp.exp((duv - r_ref[0, 0]) / t_ref[0, 0]) + 1.0)


def _tc_tail(negl2d, r2d, t2d):
    return pl.pallas_call(
        _tc_body,
        out_shape=jax.ShapeDtypeStruct(negl2d.shape, jnp.float32),
        in_specs=[
            pl.BlockSpec(memory_space=pltpu.VMEM),
            pl.BlockSpec(memory_space=pltpu.SMEM),
            pl.BlockSpec(memory_space=pltpu.SMEM),
        ],
        out_specs=pl.BlockSpec(memory_space=pltpu.VMEM),
    )(negl2d, r2d, t2d)


def kernel(u, v, theta, r, t):
    u3 = u.astype(jnp.int32).reshape(NUM_WORKERS, N_CHUNKS, CHUNK)
    v3 = v.astype(jnp.int32).reshape(NUM_WORKERS, N_CHUNKS, CHUNK)
    negl = _sc_lorentz(theta.T, u3, v3)
    r2d = jnp.asarray(r, jnp.float32).reshape(1, 1)
    t2d = jnp.asarray(t, jnp.float32).reshape(1, 1)
    out = _tc_tail(negl.reshape(128, 128), r2d, t2d)
    return out.reshape(BATCH, 1)


# speedup vs baseline: 3.3386x; 3.3386x over previous
import functools

import jax
import jax.numpy as jnp
from jax import lax
from jax.experimental import pallas as pl
from jax.experimental.pallas import tpu as pltpu
from jax.experimental.pallas import tpu_sc as plsc

NUM_ITEMS_ = 1000000
BATCH = 16384
DIM = 32
NUM_CORES = 2
NUM_SUBCORES = 16
NUM_WORKERS = NUM_CORES * NUM_SUBCORES
B_PER_W = BATCH // NUM_WORKERS
CHUNK = 128
N_CHUNKS = B_PER_W // CHUNK
LANES = 16


def _sc_body(theta_hbm, u_hbm, v_hbm, out_hbm,
             idx_u, idx_v, rows_u, rows_v, lsp_v, sem):
    wid = lax.axis_index("s") * NUM_CORES + lax.axis_index("c")
    base = wid * B_PER_W

    pltpu.sync_copy(u_hbm.at[wid], idx_u)
    pltpu.sync_copy(v_hbm.at[wid], idx_v)

    copies = []
    for j in range(N_CHUNKS):
        copies.append(pltpu.async_copy(
            theta_hbm.at[idx_u.at[j]], rows_u.at[pl.ds(j * CHUNK, CHUNK)], sem))
        copies.append(pltpu.async_copy(
            theta_hbm.at[idx_v.at[j]], rows_v.at[pl.ds(j * CHUNK, CHUNK)], sem))
    for c in copies:
        c.wait()

    lane = lax.iota(jnp.int32, LANES)

    def body(i, carry):
        rvec = i * LANES + lane
        d0 = jnp.zeros((LANES,), jnp.int32)
        acc = (plsc.load_gather(rows_u, [rvec, d0]) *
               plsc.load_gather(rows_v, [rvec, d0]))
        for d in range(1, DIM):
            dv = jnp.full((LANES,), d, jnp.int32)
            acc = acc - (plsc.load_gather(rows_u, [rvec, dv]) *
                         plsc.load_gather(rows_v, [rvec, dv]))
        lsp_v[pl.ds(i * LANES, LANES)] = acc
        return carry

    lax.fori_loop(0, B_PER_W // LANES, body, 0)

    pltpu.sync_copy(lsp_v, out_hbm.at[pl.ds(base, B_PER_W)])


def _sc_lorentz(theta_rm, u3, v3):
    mesh = plsc.VectorSubcoreMesh(core_axis_name="c", subcore_axis_name="s")
    k = pl.kernel(
        _sc_body,
        out_type=jax.ShapeDtypeStruct((BATCH,), jnp.float32),
        mesh=mesh,
        compiler_params=pltpu.CompilerParams(
            needs_layout_passes=False, use_tc_tiling_on_sc=False),
        scratch_types=[
            pltpu.VMEM((N_CHUNKS, CHUNK), jnp.int32),
            pltpu.VMEM((N_CHUNKS, CHUNK), jnp.int32),
            pltpu.VMEM((B_PER_W, DIM), jnp.float32),
            pltpu.VMEM((B_PER_W, DIM), jnp.float32),
            pltpu.VMEM((B_PER_W,), jnp.float32),
            pltpu.SemaphoreType.DMA,
        ],
    )
    return k(theta_rm, u3, v3)


def _tc_body(negl_ref, r_ref, t_ref, o_ref):
    w = jnp.clip(negl_ref[...], 1.0 + 1e-6, 100.0)
    duv = jnp.log(w + jnp.sqrt((w - 1.0) * (w + 1.0)))
    o_ref[...] = 1.0 / (jnp.exp((duv - r_ref[0, 0]) / t_ref[0, 0]) + 1.0)


def _tc_tail(negl2d, r2d, t2d):
    return pl.pallas_call(
        _tc_body,
        out_shape=jax.ShapeDtypeStruct(negl2d.shape, jnp.float32),
        in_specs=[
            pl.BlockSpec(memory_space=pltpu.VMEM),
            pl.BlockSpec(memory_space=pltpu.SMEM),
            pl.BlockSpec(memory_space=pltpu.SMEM),
        ],
        out_specs=pl.BlockSpec(memory_space=pltpu.VMEM),
    )(negl2d, r2d, t2d)


def kernel(u, v, theta, r, t):
    u3 = u.astype(jnp.int32).reshape(NUM_WORKERS, N_CHUNKS, CHUNK)
    v3 = v.astype(jnp.int32).reshape(NUM_WORKERS, N_CHUNKS, CHUNK)
    eye = jnp.eye(DIM, dtype=jnp.float32)
    theta_rm = jax.lax.dot(theta, eye,
                           precision=jax.lax.Precision.HIGHEST)
    negl = _sc_lorentz(theta_rm, u3, v3)
    r2d = jnp.asarray(r, jnp.float32).reshape(1, 1)
    t2d = jnp.asarray(t, jnp.float32).reshape(1, 1)
    out = _tc_tail(negl.reshape(128, 128), r2d, t2d)
    return out.reshape(BATCH, 1)


# speedup vs baseline: 4.9204x; 1.4738x over previous
import functools

import jax
import jax.numpy as jnp
from jax import lax
from jax.experimental import pallas as pl
from jax.experimental.pallas import tpu as pltpu
from jax.experimental.pallas import tpu_sc as plsc

NUM_ITEMS_ = 1000000
BATCH = 16384
DIM = 32
PACK = 4
NUM_LINES = NUM_ITEMS_ // PACK
NUM_CORES = 2
NUM_SUBCORES = 16
NUM_WORKERS = NUM_CORES * NUM_SUBCORES
B_PER_W = BATCH // NUM_WORKERS
CHUNK = 128
N_CHUNKS = B_PER_W // CHUNK
LANES = 16


def _sc_body(theta4_hbm, u_hbm, v_hbm, out_hbm,
             idx_u, idx_v, q_u, q_v, r32_u, r32_v,
             rows_u, rows_v, acc_v, sem):
    wid = lax.axis_index("s") * NUM_CORES + lax.axis_index("c")
    base = wid * B_PER_W

    pltpu.sync_copy(u_hbm.at[wid], idx_u)
    pltpu.sync_copy(v_hbm.at[wid], idx_v)

    for c in range(N_CHUNKS):
        def prep(s, carry):
            sl = pl.ds(s * LANES, LANES)
            uvec = idx_u[c, sl]
            vvec = idx_v[c, sl]
            q_u[c, sl] = lax.shift_right_logical(uvec, 2)
            q_v[c, sl] = lax.shift_right_logical(vvec, 2)
            r32_u[c, sl] = lax.shift_left(jnp.bitwise_and(uvec, 3), 5)
            r32_v[c, sl] = lax.shift_left(jnp.bitwise_and(vvec, 3), 5)
            return carry
        lax.fori_loop(0, CHUNK // LANES, prep, 0)

    def fire(c, buf):
        return [
            pltpu.async_copy(theta4_hbm.at[q_u.at[c]], rows_u.at[buf], sem),
            pltpu.async_copy(theta4_hbm.at[q_v.at[c]], rows_v.at[buf], sem),
        ]

    lane = lax.iota(jnp.int32, LANES)
    pending = fire(0, 0)
    for c in range(N_CHUNKS):
        nxt = fire(c + 1, (c + 1) % 2) if c + 1 < N_CHUNKS else []
        for cp in pending:
            cp.wait()
        pending = nxt
        bvec = jnp.full((LANES,), c % 2, jnp.int32)

        def grp(g, carry, c=c, bvec=bvec):
            sl = pl.ds(g * LANES, LANES)
            rvec = g * LANES + lane
            r32u = r32_u[c, sl]
            r32v = r32_v[c, sl]
            acc = (plsc.load_gather(rows_u, [bvec, rvec, r32u]) *
                   plsc.load_gather(rows_v, [bvec, rvec, r32v]))
            for d in range(1, DIM):
                acc = acc - (plsc.load_gather(rows_u, [bvec, rvec, r32u + d]) *
                             plsc.load_gather(rows_v, [bvec, rvec, r32v + d]))
            acc_v[pl.ds(c * CHUNK + g * LANES, LANES)] = acc
            return carry

        lax.fori_loop(0, CHUNK // LANES, grp, 0)

    pltpu.sync_copy(acc_v, out_hbm.at[pl.ds(base, B_PER_W)])


def _sc_lorentz(theta4, u3, v3):
    mesh = plsc.VectorSubcoreMesh(core_axis_name="c", subcore_axis_name="s")
    k = pl.kernel(
        _sc_body,
        out_type=jax.ShapeDtypeStruct((BATCH,), jnp.float32),
        mesh=mesh,
        compiler_params=pltpu.CompilerParams(
            needs_layout_passes=False, use_tc_tiling_on_sc=True),
        scratch_types=[
            pltpu.VMEM((N_CHUNKS, CHUNK), jnp.int32),
            pltpu.VMEM((N_CHUNKS, CHUNK), jnp.int32),
            pltpu.VMEM((N_CHUNKS, CHUNK), jnp.int32),
            pltpu.VMEM((N_CHUNKS, CHUNK), jnp.int32),
            pltpu.VMEM((N_CHUNKS, CHUNK), jnp.int32),
            pltpu.VMEM((N_CHUNKS, CHUNK), jnp.int32),
            pltpu.VMEM((2, CHUNK, 128), jnp.float32),
            pltpu.VMEM((2, CHUNK, 128), jnp.float32),
            pltpu.VMEM((B_PER_W,), jnp.float32),
            pltpu.SemaphoreType.DMA,
        ],
    )
    return k(theta4, u3, v3)


def _tc_body(negl_ref, r_ref, t_ref, o_ref):
    w = jnp.clip(negl_ref[...], 1.0 + 1e-6, 100.0)
    duv = jnp.log(w + jnp.sqrt((w - 1.0) * (w + 1.0)))
    o_ref[...] = 1.0 / (jnp.exp((duv - r_ref[0, 0]) / t_ref[0, 0]) + 1.0)


def _tc_tail(negl2d, r2d, t2d):
    return pl.pallas_call(
        _tc_body,
        out_shape=jax.ShapeDtypeStruct(negl2d.shape, jnp.float32),
        in_specs=[
            pl.BlockSpec(memory_space=pltpu.VMEM),
            pl.BlockSpec(memory_space=pltpu.SMEM),
            pl.BlockSpec(memory_space=pltpu.SMEM),
        ],
        out_specs=pl.BlockSpec(memory_space=pltpu.VMEM),
    )(negl2d, r2d, t2d)


def kernel(u, v, theta, r, t):
    u3 = u.astype(jnp.int32).reshape(NUM_WORKERS, N_CHUNKS, CHUNK)
    v3 = v.astype(jnp.int32).reshape(NUM_WORKERS, N_CHUNKS, CHUNK)
    theta4 = theta.reshape(NUM_LINES, PACK * DIM)
    negl = _sc_lorentz(theta4, u3, v3)
    r2d = jnp.asarray(r, jnp.float32).reshape(1, 1)
    t2d = jnp.asarray(t, jnp.float32).reshape(1, 1)
    out = _tc_tail(negl.reshape(128, 128), r2d, t2d)
    return out.reshape(BATCH, 1)


# speedup vs baseline: 4.9514x; 1.0063x over previous
import functools

import jax
import jax.numpy as jnp
from jax import lax
from jax.experimental import pallas as pl
from jax.experimental.pallas import tpu as pltpu
from jax.experimental.pallas import tpu_sc as plsc

NUM_ITEMS_ = 1000000
BATCH = 16384
DIM = 32
NUM_CORES = 2
NUM_SUBCORES = 16
NUM_WORKERS = NUM_CORES * NUM_SUBCORES
B_PER_W = BATCH // NUM_WORKERS
CHUNK = 128
N_CHUNKS = B_PER_W // CHUNK
LANES = 16


def _sc_body(theta_hbm, u_hbm, v_hbm, out_hbm,
             idx_u, idx_v, rows_u, rows_v, lsp_v, sem):
    wid = lax.axis_index("s") * NUM_CORES + lax.axis_index("c")
    base = wid * B_PER_W

    pltpu.sync_copy(u_hbm.at[wid], idx_u)
    pltpu.sync_copy(v_hbm.at[wid], idx_v)

    copies = []
    for j in range(N_CHUNKS):
        copies.append(pltpu.async_copy(
            theta_hbm.at[idx_u.at[j]], rows_u.at[pl.ds(j * CHUNK, CHUNK)], sem))
        copies.append(pltpu.async_copy(
            theta_hbm.at[idx_v.at[j]], rows_v.at[pl.ds(j * CHUNK, CHUNK)], sem))
    for c in copies:
        c.wait()

    lane = lax.iota(jnp.int32, LANES)

    def body(i, carry):
        rvec = i * LANES + lane
        d0 = jnp.zeros((LANES,), jnp.int32)
        acc = (plsc.load_gather(rows_u, [rvec, d0]) *
               plsc.load_gather(rows_v, [rvec, d0]))
        for d in range(1, DIM):
            dv = jnp.full((LANES,), d, jnp.int32)
            acc = acc - (plsc.load_gather(rows_u, [rvec, dv]) *
                         plsc.load_gather(rows_v, [rvec, dv]))
        lsp_v[pl.ds(i * LANES, LANES)] = acc
        return carry

    lax.fori_loop(0, B_PER_W // LANES, body, 0)

    pltpu.sync_copy(lsp_v, out_hbm.at[pl.ds(base, B_PER_W)])


def _sc_lorentz(theta_rm, u3, v3):
    mesh = plsc.VectorSubcoreMesh(core_axis_name="c", subcore_axis_name="s")
    k = pl.kernel(
        _sc_body,
        out_type=jax.ShapeDtypeStruct((BATCH,), jnp.float32),
        mesh=mesh,
        compiler_params=pltpu.CompilerParams(
            needs_layout_passes=False, use_tc_tiling_on_sc=False),
        scratch_types=[
            pltpu.VMEM((N_CHUNKS, CHUNK), jnp.int32),
            pltpu.VMEM((N_CHUNKS, CHUNK), jnp.int32),
            pltpu.VMEM((B_PER_W, DIM), jnp.float32),
            pltpu.VMEM((B_PER_W, DIM), jnp.float32),
            pltpu.VMEM((B_PER_W,), jnp.float32),
            pltpu.SemaphoreType.DMA,
        ],
    )
    return k(theta_rm, u3, v3)


def _tc_body(negl_ref, r_ref, t_ref, o_ref):
    w = jnp.clip(negl_ref[...], 1.0 + 1e-6, 100.0)
    duv = jnp.log(w + jnp.sqrt((w - 1.0) * (w + 1.0)))
    o_ref[...] = 1.0 / (jnp.exp((duv - r_ref[0, 0]) / t_ref[0, 0]) + 1.0)


def _tc_tail(negl2d, r2d, t2d):
    return pl.pallas_call(
        _tc_body,
        out_shape=jax.ShapeDtypeStruct(negl2d.shape, jnp.float32),
        in_specs=[
            pl.BlockSpec(memory_space=pltpu.VMEM),
            pl.BlockSpec(memory_space=pltpu.SMEM),
            pl.BlockSpec(memory_space=pltpu.SMEM),
        ],
        out_specs=pl.BlockSpec(memory_space=pltpu.VMEM),
    )(negl2d, r2d, t2d)


def kernel(u, v, theta, r, t):
    u3 = u.astype(jnp.int32).reshape(NUM_WORKERS, N_CHUNKS, CHUNK)
    v3 = v.astype(jnp.int32).reshape(NUM_WORKERS, N_CHUNKS, CHUNK)
    negl = _sc_lorentz(theta, u3, v3)
    r2d = jnp.asarray(r, jnp.float32).reshape(1, 1)
    t2d = jnp.asarray(t, jnp.float32).reshape(1, 1)
    out = _tc_tail(negl.reshape(128, 128), r2d, t2d)
    return out.reshape(BATCH, 1)


# speedup vs baseline: 5.0451x; 1.0189x over previous
import functools

import jax
import jax.numpy as jnp
from jax import lax
from jax.experimental import pallas as pl
from jax.experimental.pallas import tpu as pltpu
from jax.experimental.pallas import tpu_sc as plsc

NUM_ITEMS_ = 1000000
BATCH = 16384
DIM = 32
LINE = 128
NUM_CORES = 2
NUM_SUBCORES = 16
NUM_WORKERS = NUM_CORES * NUM_SUBCORES
B_PER_W = BATCH // NUM_WORKERS
CHUNK = 128
N_CHUNKS = B_PER_W // CHUNK
LANES = 16


def _sc_body(theta_hbm, u_hbm, v_hbm, out_hbm,
             idx_u, idx_v, rows_u, rows_v, acc_v, sem):
    wid = lax.axis_index("s") * NUM_CORES + lax.axis_index("c")
    base = wid * B_PER_W

    pltpu.sync_copy(u_hbm.at[wid], idx_u)
    pltpu.sync_copy(v_hbm.at[wid], idx_v)

    def fire(c, buf):
        return [
            pltpu.async_copy(theta_hbm.at[idx_u.at[c]], rows_u.at[buf], sem),
            pltpu.async_copy(theta_hbm.at[idx_v.at[c]], rows_v.at[buf], sem),
        ]

    lane = lax.iota(jnp.int32, LANES)
    pending = fire(0, 0)
    for c in range(N_CHUNKS):
        nxt = fire(c + 1, (c + 1) % 2) if c + 1 < N_CHUNKS else []
        for cp in pending:
            cp.wait()
        pending = nxt
        bvec = jnp.full((LANES,), c % 2, jnp.int32)

        def grp(g, carry, c=c, bvec=bvec):
            rvec = g * LANES + lane
            d0 = jnp.zeros((LANES,), jnp.int32)
            acc = (plsc.load_gather(rows_u, [bvec, rvec, d0]) *
                   plsc.load_gather(rows_v, [bvec, rvec, d0]))
            for d in range(1, DIM):
                dv = jnp.full((LANES,), d, jnp.int32)
                acc = acc - (plsc.load_gather(rows_u, [bvec, rvec, dv]) *
                             plsc.load_gather(rows_v, [bvec, rvec, dv]))
            acc_v[pl.ds(c * CHUNK + g * LANES, LANES)] = acc
            return carry

        lax.fori_loop(0, CHUNK // LANES, grp, 0)

    pltpu.sync_copy(acc_v, out_hbm.at[pl.ds(base, B_PER_W)])


def _sc_lorentz(theta128, u3, v3):
    mesh = plsc.VectorSubcoreMesh(core_axis_name="c", subcore_axis_name="s")
    k = pl.kernel(
        _sc_body,
        out_type=jax.ShapeDtypeStruct((BATCH,), jnp.float32),
        mesh=mesh,
        compiler_params=pltpu.CompilerParams(
            needs_layout_passes=False, use_tc_tiling_on_sc=True),
        scratch_types=[
            pltpu.VMEM((N_CHUNKS, CHUNK), jnp.int32),
            pltpu.VMEM((N_CHUNKS, CHUNK), jnp.int32),
            pltpu.VMEM((2, CHUNK, LINE), jnp.float32),
            pltpu.VMEM((2, CHUNK, LINE), jnp.float32),
            pltpu.VMEM((B_PER_W,), jnp.float32),
            pltpu.SemaphoreType.DMA,
        ],
    )
    return k(theta128, u3, v3)


def _tc_body(negl_ref, r_ref, t_ref, o_ref):
    w = jnp.clip(negl_ref[...], 1.0 + 1e-6, 100.0)
    duv = jnp.log(w + jnp.sqrt((w - 1.0) * (w + 1.0)))
    o_ref[...] = 1.0 / (jnp.exp((duv - r_ref[0, 0]) / t_ref[0, 0]) + 1.0)


def _tc_tail(negl2d, r2d, t2d):
    return pl.pallas_call(
        _tc_body,
        out_shape=jax.ShapeDtypeStruct(negl2d.shape, jnp.float32),
        in_specs=[
            pl.BlockSpec(memory_space=pltpu.VMEM),
            pl.BlockSpec(memory_space=pltpu.SMEM),
            pl.BlockSpec(memory_space=pltpu.SMEM),
        ],
        out_specs=pl.BlockSpec(memory_space=pltpu.VMEM),
    )(negl2d, r2d, t2d)


def kernel(u, v, theta, r, t):
    u3 = u.astype(jnp.int32).reshape(NUM_WORKERS, N_CHUNKS, CHUNK)
    v3 = v.astype(jnp.int32).reshape(NUM_WORKERS, N_CHUNKS, CHUNK)
    theta128 = jnp.pad(theta, ((0, 0), (0, LINE - DIM)))
    negl = _sc_lorentz(theta128, u3, v3)
    r2d = jnp.asarray(r, jnp.float32).reshape(1, 1)
    t2d = jnp.asarray(t, jnp.float32).reshape(1, 1)
    out = _tc_tail(negl.reshape(128, 128), r2d, t2d)
    return out.reshape(BATCH, 1)
